# baseline (speedup 1.0000x reference)
import functools

import jax
import jax.numpy as jnp
from jax import lax
from jax.experimental import pallas as pl
from jax.experimental.pallas import tpu as pltpu
from jax.experimental.pallas import tpu_sc as plsc


def _proj_body(emb_ref, w_ref, b_ref, out_ref):
    y = lax.dot_general(
        emb_ref[...].astype(jnp.bfloat16), w_ref[...].astype(jnp.bfloat16),
        (((1,), (1,)), ((), ())),
        preferred_element_type=jnp.float32,
    )
    out_ref[...] = y + b_ref[...]


def _project_table(emb, W, b):
    V, H = emb.shape
    BLOCK = 1024
    grid = (V + BLOCK - 1) // BLOCK
    return pl.pallas_call(
        _proj_body,
        grid=(grid,),
        in_specs=[
            pl.BlockSpec((BLOCK, H), lambda i: (i, 0)),
            pl.BlockSpec((H, H), lambda i: (0, 0)),
            pl.BlockSpec((1, H), lambda i: (0, 0)),
        ],
        out_specs=pl.BlockSpec((BLOCK, H), lambda i: (i, 0)),
        out_shape=jax.ShapeDtypeStruct((V, H), jnp.float32),
    )(emb, W, b.reshape(1, H))


def _make_gather(V, H, N):
    info = plsc.get_sparse_core_info()
    NC, NS = info.num_cores, info.num_subcores
    NW = NC * NS
    assert N % NW == 0
    per_w = N // NW
    CH = 64
    assert per_w % (2 * CH) == 0
    n_iter = per_w // CH
    n_pair = n_iter // 2
    mesh = plsc.VectorSubcoreMesh(core_axis_name="c", subcore_axis_name="s")

    @functools.partial(
        pl.kernel,
        mesh=mesh,
        out_type=jax.ShapeDtypeStruct((N, H), jnp.float32),
        scratch_types=[
            pltpu.VMEM((n_iter, CH), jnp.int32),
            pltpu.VMEM((CH, H), jnp.float32),
            pltpu.VMEM((CH, H), jnp.float32),
            pltpu.SemaphoreType.DMA,
            pltpu.SemaphoreType.DMA,
        ],
    )
    def gather_k(table_hbm, idx_hbm, out_hbm, idx_all, rows0, rows1,
                 sem0, sem1):
        wid = lax.axis_index("s") * NC + lax.axis_index("c")
        base = wid * per_w
        pltpu.sync_copy(idx_hbm.at[wid], idx_all)

        def start_gather(j, rows_v, sem):
            pltpu.async_copy(table_hbm.at[idx_all.at[j]], rows_v, sem)

        def wait_gather(j, rows_v, sem):
            pltpu.make_async_copy(
                table_hbm.at[idx_all.at[j]], rows_v, sem).wait()

        def writeback(j, rows_v):
            pltpu.sync_copy(rows_v, out_hbm.at[pl.ds(base + j * CH, CH)])

        start_gather(0, rows0, sem0)

        def body(g, carry):
            j0 = 2 * g
            j1 = j0 + 1
            start_gather(j1, rows1, sem1)
            wait_gather(j0, rows0, sem0)
            writeback(j0, rows0)

            @pl.when(g + 1 < n_pair)
            def _():
                start_gather(j0 + 2, rows0, sem0)

            wait_gather(j1, rows1, sem1)
            writeback(j1, rows1)
            return carry

        lax.fori_loop(0, n_pair, body, 0)

    return gather_k, n_iter, CH


def kernel(input_ids, attention_mask, emb, W, b):
    B, L = input_ids.shape
    V, H = emb.shape
    table = _project_table(emb, W, b)
    gather_k, n_iter, CH = _make_gather(V, H, B * L)
    ids = input_ids.astype(jnp.int32).reshape(-1, n_iter, CH)
    flat = gather_k(table, ids)
    x = flat.reshape(B, L, H)
    pad_mask = attention_mask == 0
    return (x, pad_mask)


# speedup vs baseline: 1.0172x; 1.0172x over previous
import functools

import jax
import jax.numpy as jnp
from jax import lax
from jax.experimental import pallas as pl
from jax.experimental.pallas import tpu as pltpu
from jax.experimental.pallas import tpu_sc as plsc


def _proj_body(emb_ref, w_ref, b_ref, out_ref):
    y = lax.dot_general(
        emb_ref[...].astype(jnp.bfloat16), w_ref[...].astype(jnp.bfloat16),
        (((1,), (1,)), ((), ())),
        preferred_element_type=jnp.float32,
    )
    out_ref[...] = y + b_ref[...]


def _project_table(emb, W, b):
    V, H = emb.shape
    BLOCK = 2048
    grid = (V + BLOCK - 1) // BLOCK
    return pl.pallas_call(
        _proj_body,
        grid=(grid,),
        in_specs=[
            pl.BlockSpec((BLOCK, H), lambda i: (i, 0)),
            pl.BlockSpec((H, H), lambda i: (0, 0)),
            pl.BlockSpec((1, H), lambda i: (0, 0)),
        ],
        out_specs=pl.BlockSpec((BLOCK, H), lambda i: (i, 0)),
        out_shape=jax.ShapeDtypeStruct((V, H), jnp.float32),
    )(emb, W, b.reshape(1, H))


def _make_gather(V, H, N):
    info = plsc.get_sparse_core_info()
    NC, NS = info.num_cores, info.num_subcores
    NW = NC * NS
    assert N % NW == 0
    per_w = N // NW
    CH = 64
    assert per_w % (2 * CH) == 0
    n_iter = per_w // CH
    n_pair = n_iter // 2
    mesh = plsc.VectorSubcoreMesh(core_axis_name="c", subcore_axis_name="s")

    @functools.partial(
        pl.kernel,
        mesh=mesh,
        out_type=jax.ShapeDtypeStruct((N, H), jnp.float32),
        scratch_types=[
            pltpu.VMEM((n_iter, CH), jnp.int32),
            pltpu.VMEM((CH, H), jnp.float32),
            pltpu.VMEM((CH, H), jnp.float32),
            pltpu.SemaphoreType.DMA,
            pltpu.SemaphoreType.DMA,
        ],
    )
    def gather_k(table_hbm, idx_hbm, out_hbm, idx_all, rows0, rows1,
                 sem0, sem1):
        wid = lax.axis_index("s") * NC + lax.axis_index("c")
        base = wid * per_w
        pltpu.sync_copy(idx_hbm.at[wid], idx_all)

        def start_gather(j, rows_v, sem):
            pltpu.async_copy(table_hbm.at[idx_all.at[j]], rows_v, sem)

        def wait_gather(j, rows_v, sem):
            pltpu.make_async_copy(
                table_hbm.at[idx_all.at[j]], rows_v, sem).wait()

        def writeback(j, rows_v):
            pltpu.sync_copy(rows_v, out_hbm.at[pl.ds(base + j * CH, CH)])

        start_gather(0, rows0, sem0)

        def body(g, carry):
            j0 = 2 * g
            j1 = j0 + 1
            start_gather(j1, rows1, sem1)
            wait_gather(j0, rows0, sem0)
            writeback(j0, rows0)

            @pl.when(g + 1 < n_pair)
            def _():
                start_gather(j0 + 2, rows0, sem0)

            wait_gather(j1, rows1, sem1)
            writeback(j1, rows1)
            return carry

        lax.fori_loop(0, n_pair, body, 0)

    return gather_k, n_iter, CH


def kernel(input_ids, attention_mask, emb, W, b):
    B, L = input_ids.shape
    V, H = emb.shape
    table = _project_table(emb, W, b)
    gather_k, n_iter, CH = _make_gather(V, H, B * L)
    ids = input_ids.astype(jnp.int32).reshape(-1, n_iter, CH)
    flat = gather_k(table, ids)
    x = flat.reshape(B, L, H)
    pad_mask = attention_mask == 0
    return (x, pad_mask)


# speedup vs baseline: 1.0202x; 1.0029x over previous
import functools

import jax
import jax.numpy as jnp
from jax import lax
from jax.experimental import pallas as pl
from jax.experimental.pallas import tpu as pltpu
from jax.experimental.pallas import tpu_sc as plsc


def _proj_body(emb_ref, w_ref, b_ref, out_ref):
    y = lax.dot_general(
        emb_ref[...].astype(jnp.bfloat16), w_ref[...].astype(jnp.bfloat16),
        (((1,), (1,)), ((), ())),
        preferred_element_type=jnp.float32,
    )
    out_ref[...] = y + b_ref[...]


def _project_table(emb, W, b):
    V, H = emb.shape
    BLOCK = 3072
    grid = (V + BLOCK - 1) // BLOCK
    return pl.pallas_call(
        _proj_body,
        grid=(grid,),
        in_specs=[
            pl.BlockSpec((BLOCK, H), lambda i: (i, 0)),
            pl.BlockSpec((H, H), lambda i: (0, 0)),
            pl.BlockSpec((1, H), lambda i: (0, 0)),
        ],
        out_specs=pl.BlockSpec((BLOCK, H), lambda i: (i, 0)),
        out_shape=jax.ShapeDtypeStruct((V, H), jnp.float32),
    )(emb, W, b.reshape(1, H))


def _make_gather(V, H, N):
    info = plsc.get_sparse_core_info()
    NC, NS = info.num_cores, info.num_subcores
    NW = NC * NS
    assert N % NW == 0
    per_w = N // NW
    CH = 64
    assert per_w % (2 * CH) == 0
    n_iter = per_w // CH
    n_pair = n_iter // 2
    mesh = plsc.VectorSubcoreMesh(core_axis_name="c", subcore_axis_name="s")

    @functools.partial(
        pl.kernel,
        mesh=mesh,
        out_type=jax.ShapeDtypeStruct((N, H), jnp.float32),
        scratch_types=[
            pltpu.VMEM((n_iter, CH), jnp.int32),
            pltpu.VMEM((CH, H), jnp.float32),
            pltpu.VMEM((CH, H), jnp.float32),
            pltpu.SemaphoreType.DMA,
            pltpu.SemaphoreType.DMA,
        ],
    )
    def gather_k(table_hbm, idx_hbm, out_hbm, idx_all, rows0, rows1,
                 sem0, sem1):
        wid = lax.axis_index("s") * NC + lax.axis_index("c")
        base = wid * per_w
        pltpu.sync_copy(idx_hbm.at[wid], idx_all)

        def start_gather(j, rows_v, sem):
            pltpu.async_copy(table_hbm.at[idx_all.at[j]], rows_v, sem)

        def wait_gather(j, rows_v, sem):
            pltpu.make_async_copy(
                table_hbm.at[idx_all.at[j]], rows_v, sem).wait()

        def writeback(j, rows_v):
            pltpu.sync_copy(rows_v, out_hbm.at[pl.ds(base + j * CH, CH)])

        start_gather(0, rows0, sem0)

        def body(g, carry):
            j0 = 2 * g
            j1 = j0 + 1
            start_gather(j1, rows1, sem1)
            wait_gather(j0, rows0, sem0)
            writeback(j0, rows0)

            @pl.when(g + 1 < n_pair)
            def _():
                start_gather(j0 + 2, rows0, sem0)

            wait_gather(j1, rows1, sem1)
            writeback(j1, rows1)
            return carry

        lax.fori_loop(0, n_pair, body, 0)

    return gather_k, n_iter, CH


def kernel(input_ids, attention_mask, emb, W, b):
    B, L = input_ids.shape
    V, H = emb.shape
    table = _project_table(emb, W, b)
    gather_k, n_iter, CH = _make_gather(V, H, B * L)
    ids = input_ids.astype(jnp.int32).reshape(-1, n_iter, CH)
    flat = gather_k(table, ids)
    x = flat.reshape(B, L, H)
    pad_mask = attention_mask == 0
    return (x, pad_mask)
